# baseline (speedup 1.0000x reference)
import functools

import jax
import jax.numpy as jnp
from jax import lax
from jax.experimental import pallas as pl
from jax.experimental.pallas import tpu as pltpu
from jax.experimental.pallas import tpu_sc as plsc

_CLOSE = 1


def _perf_orders_body(nt, tb, f, och_ref, ord_ref, gidx_ref, perf_ref, rank_ref):
    b = pl.program_id(0)
    t = pl.program_id(1)
    x = och_ref[0]
    row = x.shape[1]
    tl = tb
    lane = lax.broadcasted_iota(jnp.int32, (tl, row), 1)
    nz = ((lane % f) == _CLOSE) & (x != 0.0)
    cand = jnp.where(nz, lane, row)
    fi = jnp.min(cand, axis=1, keepdims=True)
    fv = jnp.sum(jnp.where(lane == fi, x, 0.0), axis=1, keepdims=True)
    last = x[:, row - f + _CLOSE:row - f + _CLOSE + 1]
    perf = jnp.where(fv != 0.0, (last - fv) / fv, 0.0)
    perf_ref[pl.ds(t * tb, tb), :] = perf

    @pl.when(t == nt - 1)
    def _():
        tt = nt * tb
        pcol = perf_ref[:, :]
        prow = jnp.transpose(pcol)
        ii = lax.broadcasted_iota(jnp.int32, (tb, tt), 0)
        jj = lax.broadcasted_iota(jnp.int32, (tb, tt), 1)
        for ib in range(nt):
            pi = pcol[ib * tb:(ib + 1) * tb, :]
            gt = prow > pi
            eq = (prow == pi) & (jj < (ii + ib * tb))
            cnt = jnp.sum((gt | eq).astype(jnp.float32), axis=1, keepdims=True)
            rank_ref[pl.ds(ib * tb, tb), :] = cnt
        rrow = jnp.transpose(rank_ref[:, :])
        jf = jj.astype(jnp.float32)
        for pb in range(nt):
            pv = (ii + pb * tb).astype(jnp.float32)
            ordv = jnp.sum(jnp.where(rrow == pv, jf, 0.0), axis=1, keepdims=True)
            o = ordv.astype(jnp.int32)
            ord_ref[0, pl.ds(pb * tb, tb), :] = o
            gidx_ref[0, pl.ds(pb * tb, tb), :] = o + b * tt


def _perf_orders(och2):
    bsz, tt, row = och2.shape
    tb = 256
    nt = tt // tb
    f = 5
    body = functools.partial(_perf_orders_body, nt, tb, f)
    return pl.pallas_call(
        body,
        grid=(bsz, nt),
        in_specs=[pl.BlockSpec((1, tb, row), lambda b, t: (b, t, 0))],
        out_specs=[pl.BlockSpec((1, tt, 1), lambda b, t: (b, 0, 0)),
                   pl.BlockSpec((1, tt, 1), lambda b, t: (b, 0, 0))],
        out_shape=[jax.ShapeDtypeStruct((bsz, tt, 1), jnp.int32),
                   jax.ShapeDtypeStruct((bsz, tt, 1), jnp.int32)],
        scratch_shapes=[pltpu.VMEM((tt, 1), jnp.float32),
                        pltpu.VMEM((tt, 1), jnp.float32)],
    )(och2)


def _sc_gather(e01, oc, gidx):
    info = plsc.get_sparse_core_info()
    nc, ns = info.num_cores, info.num_subcores
    nw = nc * ns
    n, row = oc.shape
    d2 = e01.shape[1]
    rpw = n // nw
    k = 16
    nch = rpw // k
    mesh = plsc.VectorSubcoreMesh(core_axis_name="c", subcore_axis_name="s")

    @functools.partial(
        pl.kernel,
        out_type=[jax.ShapeDtypeStruct((n, d2), jnp.float32),
                  jax.ShapeDtypeStruct((n, row), jnp.float32)],
        mesh=mesh,
        scratch_types=[pltpu.VMEM((rpw,), jnp.int32),
                       pltpu.VMEM((k, d2), jnp.float32),
                       pltpu.VMEM((k, row), jnp.float32),
                       pltpu.SemaphoreType.DMA,
                       pltpu.SemaphoreType.DMA],
    )
    def gathered(e01_h, oc_h, gi_h, o01_h, o2_h, idx_v, b0, b2, s0, s2):
        wid = lax.axis_index("s") * nc + lax.axis_index("c")
        base = wid * rpw
        pltpu.sync_copy(gi_h.at[pl.ds(base, rpw)], idx_v)

        def body(c, carry):
            r0 = c * k
            ids = idx_v.at[pl.ds(r0, k)]
            c2 = pltpu.async_copy(oc_h.at[ids], b2, s2)
            c0 = pltpu.async_copy(e01_h.at[ids], b0, s0)
            c0.wait()
            c2.wait()
            pltpu.sync_copy(b0, o01_h.at[pl.ds(base + r0, k)])
            pltpu.sync_copy(b2, o2_h.at[pl.ds(base + r0, k)])
            return carry

        lax.fori_loop(0, nch, body, 0)

    return gathered(e01, oc, gidx)


def kernel(elem0, elem1, ochlv):
    bsz, tt, ll, f = ochlv.shape
    d = elem0.shape[-1]
    row = ll * f
    och2 = ochlv.reshape(bsz, tt, row)
    orders3, gidx3 = _perf_orders(och2)
    orders = orders3.reshape(bsz, tt)
    gidx = gidx3.reshape(bsz * tt)
    e01 = jnp.concatenate([elem0.reshape(bsz * tt, d),
                           elem1.reshape(bsz * tt, d)], axis=1)
    o01, o2 = _sc_gather(e01, och2.reshape(bsz * tt, row), gidx)
    return (o01[:, :d].reshape(bsz, tt, d), o01[:, d:].reshape(bsz, tt, d),
            o2.reshape(bsz, tt, ll, f), orders)


# speedup vs baseline: 4.4024x; 4.4024x over previous
import functools

import jax
import jax.numpy as jnp
from jax import lax
from jax.experimental import pallas as pl
from jax.experimental.pallas import tpu as pltpu
from jax.experimental.pallas import tpu_sc as plsc

_CLOSE = 1


def _perf_orders_body(nt, tb, closes_ref, ord_ref, perf_ref, rank_ref):
    b = pl.program_id(0)
    t = pl.program_id(1)
    x = closes_ref[0, 0]
    ll = x.shape[1]
    lane = lax.broadcasted_iota(jnp.int32, (tb, ll), 1)
    nz = x != 0.0
    cand = jnp.where(nz, lane, ll)
    fi = jnp.min(cand, axis=1, keepdims=True)
    fv = jnp.sum(jnp.where(lane == fi, x, 0.0), axis=1, keepdims=True)
    last = x[:, ll - 1:ll]
    perf = jnp.where(fv != 0.0, (last - fv) / fv, 0.0)
    perf_ref[pl.ds(t * tb, tb), :] = perf

    @pl.when(t == nt - 1)
    def _():
        tt = nt * tb
        pcol = perf_ref[:, :]
        prow = jnp.transpose(pcol)
        ii = lax.broadcasted_iota(jnp.int32, (tb, tt), 0)
        jj = lax.broadcasted_iota(jnp.int32, (tb, tt), 1)
        for ib in range(nt):
            pi = pcol[ib * tb:(ib + 1) * tb, :]
            gt = prow > pi
            eq = (prow == pi) & (jj < (ii + ib * tb))
            cnt = jnp.sum((gt | eq).astype(jnp.float32), axis=1, keepdims=True)
            rank_ref[pl.ds(ib * tb, tb), :] = cnt
        rrow = jnp.transpose(rank_ref[:, :])
        jf = jj.astype(jnp.float32)
        for pb in range(nt):
            pv = (ii + pb * tb).astype(jnp.float32)
            ordv = jnp.sum(jnp.where(rrow == pv, jf, 0.0), axis=1, keepdims=True)
            ord_ref[0, pl.ds(pb * tb, tb), :] = ordv.astype(jnp.int32)


def _perf_orders(och_t):
    bsz, f, tt, ll = och_t.shape
    tb = 256
    nt = tt // tb
    body = functools.partial(_perf_orders_body, nt, tb)
    return pl.pallas_call(
        body,
        grid=(bsz, nt),
        in_specs=[pl.BlockSpec((1, 1, tb, ll), lambda b, t: (b, _CLOSE, t, 0))],
        out_specs=[pl.BlockSpec((1, tt, 1), lambda b, t: (b, 0, 0))],
        out_shape=[jax.ShapeDtypeStruct((bsz, tt, 1), jnp.int32)],
        scratch_shapes=[pltpu.VMEM((tt, 1), jnp.float32),
                        pltpu.VMEM((tt, 1), jnp.float32)],
    )(och_t)


def _sc_gather(e01, oc2d, ordf, bsz, tt):
    info = plsc.get_sparse_core_info()
    nc, ns = info.num_cores, info.num_subcores
    nw = nc * ns
    npl, ll = oc2d.shape
    nf = npl // (bsz * tt)
    d2 = e01.shape[1]
    n = bsz * tt
    rpw = n // nw
    k = 16
    nch = rpw // k
    mesh = plsc.VectorSubcoreMesh(core_axis_name="c", subcore_axis_name="s")

    @functools.partial(
        pl.kernel,
        out_type=[jax.ShapeDtypeStruct((n, d2), jnp.float32),
                  jax.ShapeDtypeStruct((npl, ll), jnp.float32)],
        mesh=mesh,
        scratch_types=[pltpu.VMEM((rpw,), jnp.int32),
                       pltpu.VMEM((k, d2), jnp.float32),
                       pltpu.VMEM((k, d2), jnp.float32),
                       pltpu.VMEM((nf * k, ll), jnp.float32),
                       pltpu.VMEM((nf * k, ll), jnp.float32),
                       pltpu.SemaphoreType.DMA,
                       pltpu.SemaphoreType.DMA,
                       pltpu.SemaphoreType.DMA,
                       pltpu.SemaphoreType.DMA],
    )
    def gathered(e01_h, oc_h, ord_h, o01_h, o2_h,
                 idx_v, b0a, b0b, b2a, b2b, s0a, s0b, s2a, s2b):
        wid = lax.axis_index("s") * nc + lax.axis_index("c")
        base = wid * rpw
        b = base // tt
        t0 = base - b * tt
        pltpu.sync_copy(ord_h.at[pl.ds(base, rpw)], idx_v)
        bufs = ((b0a, b2a, s0a, s2a), (b0b, b2b, s0b, s2b))

        def start(c, ph):
            b0, b2, s0, s2 = bufs[ph]
            ordreg = idx_v[pl.ds(c * k, k)]
            descs = [pltpu.async_copy(e01_h.at[ordreg + b * tt], b0, s0)]
            for f in range(nf):
                pidx = ordreg + (b * nf + f) * tt
                descs.append(pltpu.async_copy(
                    oc_h.at[pidx], b2.at[pl.ds(f * k, k)], s2))
            return descs

        descs = [start(0, 0), start(1, 1)]
        for c in range(nch):
            ph = c % 2
            b0, b2, _, _ = bufs[ph]
            for d_ in descs[ph]:
                d_.wait()
            pltpu.sync_copy(b0, o01_h.at[pl.ds(base + c * k, k)])
            for f in range(nf):
                orow = (b * nf + f) * tt + t0 + c * k
                pltpu.sync_copy(b2.at[pl.ds(f * k, k)],
                                o2_h.at[pl.ds(orow, k)])
            if c + 2 < nch:
                descs[ph] = start(c + 2, ph)

    return gathered(e01, oc2d, ordf)


def kernel(elem0, elem1, ochlv):
    bsz, tt, ll, f = ochlv.shape
    d = elem0.shape[-1]
    och_t = jnp.transpose(ochlv, (0, 3, 1, 2))
    orders3 = _perf_orders(och_t)[0]
    orders = orders3.reshape(bsz, tt)
    ordf = orders3.reshape(bsz * tt)
    e01 = jnp.concatenate([elem0.reshape(bsz * tt, d),
                           elem1.reshape(bsz * tt, d)], axis=1)
    o01, o2p = _sc_gather(e01, och_t.reshape(bsz * f * tt, ll), ordf, bsz, tt)
    out2 = jnp.transpose(o2p.reshape(bsz, f, tt, ll), (0, 2, 3, 1))
    return (o01[:, :d].reshape(bsz, tt, d), o01[:, d:].reshape(bsz, tt, d),
            out2, orders)
